# Initial kernel scaffold; baseline (speedup 1.0000x reference)
import jax
import jax.numpy as jnp
from jax.experimental import pallas as pl


def kernel(query, reference_points, input_flatten, input_spatial_shapes, input_level_start_index, Wv, bv, Woff, boff, Wattn, battn, Wout, bout):
    raise NotImplementedError("write your pallas kernel here")


# speedup vs baseline: 56.0714x; 56.0714x over previous
import functools

import jax
import jax.numpy as jnp
from jax import lax
from jax.experimental import pallas as pl
from jax.experimental.pallas import tpu as pltpu
from jax.experimental.pallas import tpu_sc as plsc

_B = 2
_LEN = 7681
_DM = 256
_H = 8
_L = 4
_P = 4
_D = 32

_NW = 32
_SROWS_PER_W = 482
_SROWS_PAD = _NW * _SROWS_PER_W
_NOUT = _B * _LEN * _H
_NOUT_PAD = _SROWS_PAD * _H
_STEPS = _SROWS_PER_W // 2
_G = 16
_NIDX = _G * 64
_BLK = 1928


def _proj_body(q_ref, if_ref, rp_ref, wv_ref, bv_ref, woff_ref, boffm_ref,
               e_ref, wattn_ref, battn_ref, v_ref, s_ref, a_ref):
    v_ref[...] = (jnp.dot(if_ref[...], wv_ref[...],
                          preferred_element_type=jnp.float32) + bv_ref[...])
    rp = rp_ref[...]
    rp_hi = rp.astype(jnp.bfloat16).astype(jnp.float32)
    rp_lo = rp - rp_hi
    s_ref[...] = (jnp.dot(q_ref[...], woff_ref[...],
                          preferred_element_type=jnp.float32)
                  + jnp.dot(rp_hi, e_ref[...],
                            preferred_element_type=jnp.float32)
                  + jnp.dot(rp_lo, e_ref[...],
                            preferred_element_type=jnp.float32)
                  + boffm_ref[...])
    a_ref[...] = (jnp.dot(q_ref[...], wattn_ref[...],
                          preferred_element_type=jnp.float32) + battn_ref[...])


def _out_body(x_ref, w_ref, b_ref, o_ref):
    o_ref[...] = (jnp.dot(x_ref[...], w_ref[...],
                          preferred_element_type=jnp.float32) + b_ref[...])


def _row_spec(n):
    return pl.BlockSpec((_BLK, n), lambda i: (i, 0))


def _full_spec(m, n):
    return pl.BlockSpec((m, n), lambda i: (0, 0))


def _sc_body(wh_hbm, table_hbm, s_hbm, a_hbm, out_hbm,
             whv, sbuf, abuf, idxbuf, wbuf, rowsbuf, outv, sem):
    wid = lax.axis_index("s") * 2 + lax.axis_index("c")
    row_base = wid * _SROWS_PER_W

    ar16 = lax.iota(jnp.int32, 16)

    pltpu.sync_copy(wh_hbm, whv)
    wlv = whv[0, :]
    hlv = whv[1, :]
    stv = whv[2, :]
    wlv_i = wlv.astype(jnp.int32)
    hlv_i = hlv.astype(jnp.int32)
    stv_i = stv.astype(jnp.int32)

    def step_body(step, _):
        r0 = row_base + step * 2
        o0 = r0 * _H
        pltpu.sync_copy(s_hbm.at[pl.ds(r0, 2)], sbuf)
        pltpu.sync_copy(a_hbm.at[pl.ds(r0, 2)], abuf)

        def samp_body(o_local, __):
            og = o0 + o_local
            h = lax.rem(o_local, _H)
            srow = o_local // _H
            b = jnp.where(og >= _LEN * _H, 1, 0)

            def _bfly(v, f):
                for s in (8, 4, 2, 1):
                    v = f(v, v.at[jnp.bitwise_xor(ar16, s)]
                          .get(mode="promise_in_bounds"))
                return v

            lg = abuf[srow, pl.ds(h * 16, 16)]
            m = _bfly(lg, jnp.maximum)
            e = jnp.exp(lg - m)
            aw = e / _bfly(e, lambda u, v: u + v)

            rsp = jnp.full((16,), srow, jnp.int32)
            xcol = h * 32 + 2 * ar16
            x = plsc.load_gather(sbuf, [rsp, xcol])
            y = plsc.load_gather(sbuf, [rsp, xcol + 1])

            xi = x.astype(jnp.int32)
            x0 = jnp.where(xi.astype(jnp.float32) > x, xi - 1, xi)
            fx = x - x0.astype(jnp.float32)
            yi = y.astype(jnp.int32)
            y0 = jnp.where(yi.astype(jnp.float32) > y, yi - 1, yi)
            fy = y - y0.astype(jnp.float32)

            addend = b * (_LEN * _H) + h
            for ci, (dy, dx) in enumerate(((0, 0), (0, 1), (1, 0), (1, 1))):
                xc = x0 + dx
                yc = y0 + dy
                valid = ((xc >= 0) & (xc <= wlv_i - 1)
                         & (yc >= 0) & (yc <= hlv_i - 1))
                xcc = jnp.clip(xc, 0, wlv_i - 1)
                ycc = jnp.clip(yc, 0, hlv_i - 1)
                rowi = (ycc * wlv_i + xcc + stv_i) * _H + addend
                wx = fx if dx else 1.0 - fx
                wy = fy if dy else 1.0 - fy
                wgt = jnp.where(valid, wy * wx * aw, 0.0)
                f = o_local * 64 + ci * 16
                fr = f // 128
                fc = lax.rem(f, 128)
                idxbuf[fr, pl.ds(fc, 16)] = rowi
                wbuf[fr, pl.ds(fc, 16)] = wgt
            return __

        lax.fori_loop(0, _G, samp_body, 0, unroll=False)

        copies = [pltpu.async_copy(table_hbm.at[idxbuf.at[jc]],
                                   rowsbuf.at[pl.ds(jc * 128, 128)], sem)
                  for jc in range(_NIDX // 128)]
        for c in copies:
            c.wait()

        def acc_body(o_local, __):
            def j_body(j, accs):
                a0, a1 = accs
                f = o_local * 64 + j
                w = plsc.load_gather(
                    wbuf, [jnp.full((16,), f // 128, jnp.int32),
                           jnp.full((16,), lax.rem(f, 128), jnp.int32)])
                a0 = a0 + w * rowsbuf[f, pl.ds(0, 16)]
                a1 = a1 + w * rowsbuf[f, pl.ds(16, 16)]
                return a0, a1

            z = jnp.zeros((16,), jnp.float32)
            a0, a1 = lax.fori_loop(0, 64, j_body, (z, z), unroll=4)
            outv[o_local, pl.ds(0, 16)] = a0
            outv[o_local, pl.ds(16, 16)] = a1
            return __

        lax.fori_loop(0, _G, acc_body, 0, unroll=False)
        pltpu.sync_copy(outv, out_hbm.at[pl.ds(o0, _G)])
        return _

    lax.fori_loop(0, _STEPS, step_body, 0, unroll=False)


_sc_gather = pl.kernel(
    _sc_body,
    out_type=jax.ShapeDtypeStruct((_NOUT_PAD, _D), jnp.float32),
    mesh=plsc.VectorSubcoreMesh(core_axis_name="c", subcore_axis_name="s"),
    compiler_params=pltpu.CompilerParams(needs_layout_passes=False,
                                         use_tc_tiling_on_sc=False),
    scratch_types=[
        pltpu.VMEM((3, 16), jnp.float32),
        pltpu.VMEM((2, _DM), jnp.float32),
        pltpu.VMEM((2, _H * 16), jnp.float32),
        pltpu.VMEM((_NIDX // 128, 128), jnp.int32),
        pltpu.VMEM((_NIDX // 128, 128), jnp.float32),
        pltpu.VMEM((_NIDX, _D), jnp.float32),
        pltpu.VMEM((_G, _D), jnp.float32),
        pltpu.SemaphoreType.DMA,
    ],
)


@jax.jit
def kernel(query, reference_points, input_flatten, input_spatial_shapes,
           input_level_start_index, Wv, bv, Woff, boff, Wattn, battn,
           Wout, bout):
    Bn, Lq, _ = query.shape

    q2 = query.reshape(Bn * Lq, _DM)
    if2 = input_flatten.reshape(Bn * Lq, _DM)
    rp2 = reference_points.reshape(Bn * Lq, _L * 2)
    pad = _SROWS_PAD - Bn * Lq
    q2 = jnp.pad(q2, ((0, pad), (0, 0)))
    if2 = jnp.pad(if2, ((0, pad), (0, 0)))
    rp2 = jnp.pad(rp2, ((0, pad), (0, 0)))

    wh = input_spatial_shapes.astype(jnp.float32)
    scale = jnp.stack([wh[:, 1], wh[:, 0]], axis=-1)
    lc = jnp.arange(_L * 2)
    col_l = (jnp.arange(_H * _L * _P * 2) // (2 * _P)) % _L
    col_c = jnp.arange(_H * _L * _P * 2) % 2
    sel = ((col_l[None, :] * 2 + col_c[None, :]) == lc[:, None])
    E = jnp.where(sel, scale.reshape(_L * 2)[:, None], 0.0).astype(jnp.float32)

    boffm = (boff - 0.5).reshape(1, -1)

    outs = pl.pallas_call(
        _proj_body,
        grid=(_SROWS_PAD // _BLK,),
        in_specs=[
            _row_spec(_DM), _row_spec(_DM), _row_spec(_L * 2),
            _full_spec(_DM, _DM), _full_spec(1, _DM),
            _full_spec(_DM, _DM), _full_spec(1, _DM),
            _full_spec(_L * 2, _DM),
            _full_spec(_DM, _H * 16), _full_spec(1, _H * 16),
        ],
        out_specs=[_row_spec(_DM), _row_spec(_DM), _row_spec(_H * 16)],
        out_shape=[
            jax.ShapeDtypeStruct((_SROWS_PAD, _DM), jnp.float32),
            jax.ShapeDtypeStruct((_SROWS_PAD, _DM), jnp.float32),
            jax.ShapeDtypeStruct((_SROWS_PAD, _H * 16), jnp.float32),
        ],
    )(q2, if2, rp2, Wv, bv.reshape(1, -1), Woff, boffm, E,
      Wattn, battn.reshape(1, -1))
    value, S, A = outs

    table = value[:Bn * Lq].reshape(Bn * Lq * _H, _D)

    lvl = jnp.arange(16) // _P
    whlanes = jnp.stack([
        scale[:, 0][lvl], scale[:, 1][lvl],
        input_level_start_index.astype(jnp.float32)[lvl],
    ]).astype(jnp.float32)

    acc = _sc_gather(whlanes, table, S, A)

    acc2 = jnp.pad(acc[:_NOUT].reshape(Bn * Lq, _DM), ((0, pad), (0, 0)))

    out = pl.pallas_call(
        _out_body,
        grid=(_SROWS_PAD // _BLK,),
        in_specs=[_row_spec(_DM), _full_spec(_DM, _DM), _full_spec(1, _DM)],
        out_specs=_row_spec(_DM),
        out_shape=jax.ShapeDtypeStruct((_SROWS_PAD, _DM), jnp.float32),
    )(acc2, Wout, bout.reshape(1, -1))

    return out[:Bn * Lq].reshape(Bn, Lq, _DM)


# speedup vs baseline: 60.4714x; 1.0785x over previous
import functools

import jax
import jax.numpy as jnp
from jax import lax
from jax.experimental import pallas as pl
from jax.experimental.pallas import tpu as pltpu
from jax.experimental.pallas import tpu_sc as plsc

_B = 2
_LEN = 7681
_DM = 256
_H = 8
_L = 4
_P = 4
_D = 32

_NW = 32
_SROWS_PER_W = 482
_SROWS_PAD = _NW * _SROWS_PER_W
_NOUT = _B * _LEN * _H
_NOUT_PAD = _SROWS_PAD * _H
_STEPS = _SROWS_PER_W // 2
_G = 16
_NIDX = _G * 64
_BLK = 1928


def _proj_body(q_ref, if_ref, rp_ref, wv_ref, bv_ref, woff_ref, boffm_ref,
               e_ref, wattn_ref, battn_ref, v_ref, s_ref, a_ref):
    v_ref[...] = (jnp.dot(if_ref[...], wv_ref[...],
                          preferred_element_type=jnp.float32) + bv_ref[...])
    rp = rp_ref[...]
    rp_hi = rp.astype(jnp.bfloat16).astype(jnp.float32)
    rp_lo = rp - rp_hi
    s_ref[...] = (jnp.dot(q_ref[...], woff_ref[...],
                          preferred_element_type=jnp.float32)
                  + jnp.dot(rp_hi, e_ref[...],
                            preferred_element_type=jnp.float32)
                  + jnp.dot(rp_lo, e_ref[...],
                            preferred_element_type=jnp.float32)
                  + boffm_ref[...])
    a_ref[...] = (jnp.dot(q_ref[...], wattn_ref[...],
                          preferred_element_type=jnp.float32) + battn_ref[...])


def _out_body(x_ref, w_ref, b_ref, o_ref):
    o_ref[...] = (jnp.dot(x_ref[...], w_ref[...],
                          preferred_element_type=jnp.float32) + b_ref[...])


def _row_spec(n):
    return pl.BlockSpec((_BLK, n), lambda i: (i, 0))


def _full_spec(m, n):
    return pl.BlockSpec((m, n), lambda i: (0, 0))


def _sc_body(wh_hbm, table_hbm, s_hbm, a_hbm, out_hbm,
             whv, sbuf, abuf, idxbuf, wbuf, rowsbuf, outv, sem):
    wid = lax.axis_index("s") * 2 + lax.axis_index("c")
    row_base = wid * _SROWS_PER_W

    ar16 = lax.iota(jnp.int32, 16)

    pltpu.sync_copy(wh_hbm, whv)
    wlv = whv[0, :]
    hlv = whv[1, :]
    stv = whv[2, :]
    wlv_i = wlv.astype(jnp.int32)
    hlv_i = hlv.astype(jnp.int32)
    stv_i = stv.astype(jnp.int32)

    def step_body(step, _):
        r0 = row_base + step * 2
        o0 = r0 * _H
        pltpu.sync_copy(s_hbm.at[pl.ds(r0, 2)], sbuf)
        pltpu.sync_copy(a_hbm.at[pl.ds(r0, 2)], abuf)

        def samp_body(o_local, __):
            og = o0 + o_local
            h = lax.rem(o_local, _H)
            srow = o_local // _H
            b = jnp.where(og >= _LEN * _H, 1, 0)

            def _bfly(v, f):
                for s in (8, 4, 2, 1):
                    v = f(v, v.at[jnp.bitwise_xor(ar16, s)]
                          .get(mode="promise_in_bounds"))
                return v

            lg = abuf[srow, pl.ds(h * 16, 16)]
            m = _bfly(lg, jnp.maximum)
            e = jnp.exp(lg - m)
            aw = e / _bfly(e, lambda u, v: u + v)

            rsp = jnp.full((16,), srow, jnp.int32)
            xcol = h * 32 + 2 * ar16
            x = plsc.load_gather(sbuf, [rsp, xcol])
            y = plsc.load_gather(sbuf, [rsp, xcol + 1])

            xi = x.astype(jnp.int32)
            x0 = jnp.where(xi.astype(jnp.float32) > x, xi - 1, xi)
            fx = x - x0.astype(jnp.float32)
            yi = y.astype(jnp.int32)
            y0 = jnp.where(yi.astype(jnp.float32) > y, yi - 1, yi)
            fy = y - y0.astype(jnp.float32)

            addend = b * (_LEN * _H) + h
            for ci, (dy, dx) in enumerate(((0, 0), (0, 1), (1, 0), (1, 1))):
                xc = x0 + dx
                yc = y0 + dy
                valid = ((xc >= 0) & (xc <= wlv_i - 1)
                         & (yc >= 0) & (yc <= hlv_i - 1))
                xcc = jnp.clip(xc, 0, wlv_i - 1)
                ycc = jnp.clip(yc, 0, hlv_i - 1)
                rowi = (ycc * wlv_i + xcc + stv_i) * _H + addend
                wx = fx if dx else 1.0 - fx
                wy = fy if dy else 1.0 - fy
                wgt = jnp.where(valid, wy * wx * aw, 0.0)
                f = o_local * 64 + ci * 16
                fr = f // 128
                fc = lax.rem(f, 128)
                idxbuf[fr, pl.ds(fc, 16)] = rowi
                wbuf[fr, pl.ds(fc, 16)] = wgt
            return __

        lax.fori_loop(0, _G, samp_body, 0, unroll=False)

        copies = [pltpu.async_copy(table_hbm.at[idxbuf.at[jc]],
                                   rowsbuf.at[pl.ds(jc * 128, 128)], sem)
                  for jc in range(_NIDX // 128)]
        for c in copies:
            c.wait()

        def acc_body(o_local, __):
            wvecs = []
            for u in range(4):
                f0 = o_local * 64 + u * 16
                wvecs.append(wbuf[f0 // 128, pl.ds(lax.rem(f0, 128), 16)])

            def j_body(j, accs):
                a = list(accs)
                jv = jnp.full((16,), j, jnp.int32)
                for u in range(4):
                    w = wvecs[u].at[jv].get(mode="promise_in_bounds")
                    f = o_local * 64 + u * 16 + j
                    a[2 * u] = a[2 * u] + w * rowsbuf[f, pl.ds(0, 16)]
                    a[2 * u + 1] = a[2 * u + 1] + w * rowsbuf[f, pl.ds(16, 16)]
                return tuple(a)

            z = jnp.zeros((16,), jnp.float32)
            accs = lax.fori_loop(0, 16, j_body, (z,) * 8, unroll=2)
            outv[o_local, pl.ds(0, 16)] = (accs[0] + accs[2]) + (accs[4] + accs[6])
            outv[o_local, pl.ds(16, 16)] = (accs[1] + accs[3]) + (accs[5] + accs[7])
            return __

        lax.fori_loop(0, _G, acc_body, 0, unroll=False)
        pltpu.sync_copy(outv, out_hbm.at[pl.ds(o0, _G)])
        return _

    lax.fori_loop(0, _STEPS, step_body, 0, unroll=False)


_sc_gather = pl.kernel(
    _sc_body,
    out_type=jax.ShapeDtypeStruct((_NOUT_PAD, _D), jnp.float32),
    mesh=plsc.VectorSubcoreMesh(core_axis_name="c", subcore_axis_name="s"),
    compiler_params=pltpu.CompilerParams(needs_layout_passes=False,
                                         use_tc_tiling_on_sc=False),
    scratch_types=[
        pltpu.VMEM((3, 16), jnp.float32),
        pltpu.VMEM((2, _DM), jnp.float32),
        pltpu.VMEM((2, _H * 16), jnp.float32),
        pltpu.VMEM((_NIDX // 128, 128), jnp.int32),
        pltpu.VMEM((_NIDX // 128, 128), jnp.float32),
        pltpu.VMEM((_NIDX, _D), jnp.float32),
        pltpu.VMEM((_G, _D), jnp.float32),
        pltpu.SemaphoreType.DMA,
    ],
)


@jax.jit
def kernel(query, reference_points, input_flatten, input_spatial_shapes,
           input_level_start_index, Wv, bv, Woff, boff, Wattn, battn,
           Wout, bout):
    Bn, Lq, _ = query.shape

    q2 = query.reshape(Bn * Lq, _DM)
    if2 = input_flatten.reshape(Bn * Lq, _DM)
    rp2 = reference_points.reshape(Bn * Lq, _L * 2)
    pad = _SROWS_PAD - Bn * Lq
    q2 = jnp.pad(q2, ((0, pad), (0, 0)))
    if2 = jnp.pad(if2, ((0, pad), (0, 0)))
    rp2 = jnp.pad(rp2, ((0, pad), (0, 0)))

    wh = input_spatial_shapes.astype(jnp.float32)
    scale = jnp.stack([wh[:, 1], wh[:, 0]], axis=-1)
    lc = jnp.arange(_L * 2)
    col_l = (jnp.arange(_H * _L * _P * 2) // (2 * _P)) % _L
    col_c = jnp.arange(_H * _L * _P * 2) % 2
    sel = ((col_l[None, :] * 2 + col_c[None, :]) == lc[:, None])
    E = jnp.where(sel, scale.reshape(_L * 2)[:, None], 0.0).astype(jnp.float32)

    boffm = (boff - 0.5).reshape(1, -1)

    outs = pl.pallas_call(
        _proj_body,
        grid=(_SROWS_PAD // _BLK,),
        in_specs=[
            _row_spec(_DM), _row_spec(_DM), _row_spec(_L * 2),
            _full_spec(_DM, _DM), _full_spec(1, _DM),
            _full_spec(_DM, _DM), _full_spec(1, _DM),
            _full_spec(_L * 2, _DM),
            _full_spec(_DM, _H * 16), _full_spec(1, _H * 16),
        ],
        out_specs=[_row_spec(_DM), _row_spec(_DM), _row_spec(_H * 16)],
        out_shape=[
            jax.ShapeDtypeStruct((_SROWS_PAD, _DM), jnp.float32),
            jax.ShapeDtypeStruct((_SROWS_PAD, _DM), jnp.float32),
            jax.ShapeDtypeStruct((_SROWS_PAD, _H * 16), jnp.float32),
        ],
    )(q2, if2, rp2, Wv, bv.reshape(1, -1), Woff, boffm, E,
      Wattn, battn.reshape(1, -1))
    value, S, A = outs

    table = value[:Bn * Lq].reshape(Bn * Lq * _H, _D)

    lvl = jnp.arange(16) // _P
    whlanes = jnp.stack([
        scale[:, 0][lvl], scale[:, 1][lvl],
        input_level_start_index.astype(jnp.float32)[lvl],
    ]).astype(jnp.float32)

    acc = _sc_gather(whlanes, table, S, A)

    acc2 = jnp.pad(acc[:_NOUT].reshape(Bn * Lq, _DM), ((0, pad), (0, 0)))

    out = pl.pallas_call(
        _out_body,
        grid=(_SROWS_PAD // _BLK,),
        in_specs=[_row_spec(_DM), _full_spec(_DM, _DM), _full_spec(1, _DM)],
        out_specs=_row_spec(_DM),
        out_shape=jax.ShapeDtypeStruct((_SROWS_PAD, _DM), jnp.float32),
    )(acc2, Wout, bout.reshape(1, -1))

    return out[:Bn * Lq].reshape(Bn, Lq, _DM)


# speedup vs baseline: 68.9744x; 1.1406x over previous
import functools

import jax
import jax.numpy as jnp
from jax import lax
from jax.experimental import pallas as pl
from jax.experimental.pallas import tpu as pltpu
from jax.experimental.pallas import tpu_sc as plsc

_B = 2
_LEN = 7681
_DM = 256
_H = 8
_L = 4
_P = 4
_D = 32

_NW = 32
_SROWS_PER_W = 484
_SROWS_PAD = _NW * _SROWS_PER_W
_NOUT = _B * _LEN * _H
_NOUT_PAD = _SROWS_PAD * _H
_STEPS = _SROWS_PER_W // 4
_G = 32
_NIDX = _G * 64
_BLK = _SROWS_PAD // 8


def _proj_body(q_ref, if_ref, rp_ref, wv_ref, bv_ref, woff_ref, boffm_ref,
               e_ref, wattn_ref, battn_ref, v_ref, s_ref, a_ref):
    v_ref[...] = (jnp.dot(if_ref[...], wv_ref[...],
                          preferred_element_type=jnp.float32) + bv_ref[...])
    rp = rp_ref[...]
    rp_hi = rp.astype(jnp.bfloat16).astype(jnp.float32)
    rp_lo = rp - rp_hi
    s_ref[...] = (jnp.dot(q_ref[...], woff_ref[...],
                          preferred_element_type=jnp.float32)
                  + jnp.dot(rp_hi, e_ref[...],
                            preferred_element_type=jnp.float32)
                  + jnp.dot(rp_lo, e_ref[...],
                            preferred_element_type=jnp.float32)
                  + boffm_ref[...])
    a_ref[...] = (jnp.dot(q_ref[...], wattn_ref[...],
                          preferred_element_type=jnp.float32) + battn_ref[...])


def _out_body(x_ref, w_ref, b_ref, o_ref):
    o_ref[...] = (jnp.dot(x_ref[...], w_ref[...],
                          preferred_element_type=jnp.float32) + b_ref[...])


def _row_spec(n):
    return pl.BlockSpec((_BLK, n), lambda i: (i, 0))


def _full_spec(m, n):
    return pl.BlockSpec((m, n), lambda i: (0, 0))


def _sc_body(wh_hbm, table_hbm, s_hbm, a_hbm, out_hbm,
             whv, sbuf, abuf, idxbuf, wbuf, rowsbuf, outv, sem):
    wid = lax.axis_index("s") * 2 + lax.axis_index("c")
    row_base = wid * _SROWS_PER_W

    ar16 = lax.iota(jnp.int32, 16)

    pltpu.sync_copy(wh_hbm, whv)
    wlv = whv[0, :]
    hlv = whv[1, :]
    stv = whv[2, :]
    wlv_i = wlv.astype(jnp.int32)
    hlv_i = hlv.astype(jnp.int32)
    stv_i = stv.astype(jnp.int32)

    def step_body(step, _):
        r0 = row_base + step * 4
        o0 = r0 * _H
        pltpu.sync_copy(s_hbm.at[pl.ds(r0, 4)], sbuf)
        pltpu.sync_copy(a_hbm.at[pl.ds(r0, 4)], abuf)

        def samp_body(o_local, __):
            og = o0 + o_local
            h = lax.rem(o_local, _H)
            srow = o_local // _H
            b = jnp.where(og >= _LEN * _H, 1, 0)

            def _bfly(v, f):
                for s in (8, 4, 2, 1):
                    v = f(v, v.at[jnp.bitwise_xor(ar16, s)]
                          .get(mode="promise_in_bounds"))
                return v

            lg = abuf[srow, pl.ds(h * 16, 16)]
            m = _bfly(lg, jnp.maximum)
            e = jnp.exp(lg - m)
            aw = e / _bfly(e, lambda u, v: u + v)

            rsp = jnp.full((16,), srow, jnp.int32)
            xcol = h * 32 + 2 * ar16
            x = plsc.load_gather(sbuf, [rsp, xcol])
            y = plsc.load_gather(sbuf, [rsp, xcol + 1])

            xi = x.astype(jnp.int32)
            x0 = jnp.where(xi.astype(jnp.float32) > x, xi - 1, xi)
            fx = x - x0.astype(jnp.float32)
            yi = y.astype(jnp.int32)
            y0 = jnp.where(yi.astype(jnp.float32) > y, yi - 1, yi)
            fy = y - y0.astype(jnp.float32)

            addend = b * (_LEN * _H) + h
            for ci, (dy, dx) in enumerate(((0, 0), (0, 1), (1, 0), (1, 1))):
                xc = x0 + dx
                yc = y0 + dy
                valid = ((xc >= 0) & (xc <= wlv_i - 1)
                         & (yc >= 0) & (yc <= hlv_i - 1))
                xcc = jnp.clip(xc, 0, wlv_i - 1)
                ycc = jnp.clip(yc, 0, hlv_i - 1)
                rowi = (ycc * wlv_i + xcc + stv_i) * _H + addend
                wx = fx if dx else 1.0 - fx
                wy = fy if dy else 1.0 - fy
                wgt = jnp.where(valid, wy * wx * aw, 0.0)
                f = o_local * 64 + ci * 16
                fr = f // 128
                fc = lax.rem(f, 128)
                idxbuf[fr, pl.ds(fc, 16)] = rowi
                wbuf[fr, pl.ds(fc, 16)] = wgt
            return __

        lax.fori_loop(0, _G, samp_body, 0, unroll=False)

        copies = [pltpu.async_copy(table_hbm.at[idxbuf.at[jc]],
                                   rowsbuf.at[pl.ds(jc * 128, 128)], sem)
                  for jc in range(_NIDX // 128)]
        for c in copies:
            c.wait()

        def acc_body(o_local, __):
            wvecs = []
            for u in range(4):
                f0 = o_local * 64 + u * 16
                wvecs.append(wbuf[f0 // 128, pl.ds(lax.rem(f0, 128), 16)])

            def j_body(j, accs):
                a = list(accs)
                jv = jnp.full((16,), j, jnp.int32)
                for u in range(4):
                    w = wvecs[u].at[jv].get(mode="promise_in_bounds")
                    f = o_local * 64 + u * 16 + j
                    a[2 * u] = a[2 * u] + w * rowsbuf[f, pl.ds(0, 16)]
                    a[2 * u + 1] = a[2 * u + 1] + w * rowsbuf[f, pl.ds(16, 16)]
                return tuple(a)

            z = jnp.zeros((16,), jnp.float32)
            accs = lax.fori_loop(0, 16, j_body, (z,) * 8, unroll=2)
            outv[o_local, pl.ds(0, 16)] = (accs[0] + accs[2]) + (accs[4] + accs[6])
            outv[o_local, pl.ds(16, 16)] = (accs[1] + accs[3]) + (accs[5] + accs[7])
            return __

        lax.fori_loop(0, _G, acc_body, 0, unroll=False)
        pltpu.sync_copy(outv, out_hbm.at[pl.ds(o0, _G)])
        return _

    lax.fori_loop(0, _STEPS, step_body, 0, unroll=False)


_sc_gather = pl.kernel(
    _sc_body,
    out_type=jax.ShapeDtypeStruct((_NOUT_PAD, _D), jnp.float32),
    mesh=plsc.VectorSubcoreMesh(core_axis_name="c", subcore_axis_name="s"),
    compiler_params=pltpu.CompilerParams(needs_layout_passes=False,
                                         use_tc_tiling_on_sc=False),
    scratch_types=[
        pltpu.VMEM((3, 16), jnp.float32),
        pltpu.VMEM((4, _DM), jnp.float32),
        pltpu.VMEM((4, _H * 16), jnp.float32),
        pltpu.VMEM((_NIDX // 128, 128), jnp.int32),
        pltpu.VMEM((_NIDX // 128, 128), jnp.float32),
        pltpu.VMEM((_NIDX, _D), jnp.float32),
        pltpu.VMEM((_G, _D), jnp.float32),
        pltpu.SemaphoreType.DMA,
    ],
)


@jax.jit
def kernel(query, reference_points, input_flatten, input_spatial_shapes,
           input_level_start_index, Wv, bv, Woff, boff, Wattn, battn,
           Wout, bout):
    Bn, Lq, _ = query.shape

    q2 = query.reshape(Bn * Lq, _DM)
    if2 = input_flatten.reshape(Bn * Lq, _DM)
    rp2 = reference_points.reshape(Bn * Lq, _L * 2)
    pad = _SROWS_PAD - Bn * Lq
    q2 = jnp.pad(q2, ((0, pad), (0, 0)))
    if2 = jnp.pad(if2, ((0, pad), (0, 0)))
    rp2 = jnp.pad(rp2, ((0, pad), (0, 0)))

    wh = input_spatial_shapes.astype(jnp.float32)
    scale = jnp.stack([wh[:, 1], wh[:, 0]], axis=-1)
    lc = jnp.arange(_L * 2)
    col_l = (jnp.arange(_H * _L * _P * 2) // (2 * _P)) % _L
    col_c = jnp.arange(_H * _L * _P * 2) % 2
    sel = ((col_l[None, :] * 2 + col_c[None, :]) == lc[:, None])
    E = jnp.where(sel, scale.reshape(_L * 2)[:, None], 0.0).astype(jnp.float32)

    boffm = (boff - 0.5).reshape(1, -1)

    outs = pl.pallas_call(
        _proj_body,
        grid=(_SROWS_PAD // _BLK,),
        in_specs=[
            _row_spec(_DM), _row_spec(_DM), _row_spec(_L * 2),
            _full_spec(_DM, _DM), _full_spec(1, _DM),
            _full_spec(_DM, _DM), _full_spec(1, _DM),
            _full_spec(_L * 2, _DM),
            _full_spec(_DM, _H * 16), _full_spec(1, _H * 16),
        ],
        out_specs=[_row_spec(_DM), _row_spec(_DM), _row_spec(_H * 16)],
        out_shape=[
            jax.ShapeDtypeStruct((_SROWS_PAD, _DM), jnp.float32),
            jax.ShapeDtypeStruct((_SROWS_PAD, _DM), jnp.float32),
            jax.ShapeDtypeStruct((_SROWS_PAD, _H * 16), jnp.float32),
        ],
    )(q2, if2, rp2, Wv, bv.reshape(1, -1), Woff, boffm, E,
      Wattn, battn.reshape(1, -1))
    value, S, A = outs

    table = value[:Bn * Lq].reshape(Bn * Lq * _H, _D)

    lvl = jnp.arange(16) // _P
    whlanes = jnp.stack([
        scale[:, 0][lvl], scale[:, 1][lvl],
        input_level_start_index.astype(jnp.float32)[lvl],
    ]).astype(jnp.float32)

    acc = _sc_gather(whlanes, table, S, A)

    acc2 = jnp.pad(acc[:_NOUT].reshape(Bn * Lq, _DM), ((0, pad), (0, 0)))

    out = pl.pallas_call(
        _out_body,
        grid=(_SROWS_PAD // _BLK,),
        in_specs=[_row_spec(_DM), _full_spec(_DM, _DM), _full_spec(1, _DM)],
        out_specs=_row_spec(_DM),
        out_shape=jax.ShapeDtypeStruct((_SROWS_PAD, _DM), jnp.float32),
    )(acc2, Wout, bout.reshape(1, -1))

    return out[:Bn * Lq].reshape(Bn, Lq, _DM)
